# baseline (speedup 1.0000x reference)
import functools

import jax
import jax.numpy as jnp
from jax import lax
from jax.experimental import pallas as pl
from jax.experimental.pallas import tpu as pltpu
from jax.experimental.pallas import tpu_sc as plsc

_B, _C, _K, _HW = 16, 256, 128, 1024


def _tc_body(q_ref, w_ref, idx_ref, loss_ref):
    q = q_ref[0]
    w = w_ref[...]

    ab = lax.dot_general(w, q, (((1,), (0,)), ((), ())),
                         preferred_element_type=jnp.float32)
    b2 = jnp.sum(w * w, axis=1, keepdims=True)
    a2 = jnp.sum(q * q, axis=0, keepdims=True)
    d2 = jnp.maximum(a2 + b2 - 2.0 * ab, 0.0)

    m = jnp.min(d2, axis=0, keepdims=True)
    kio = lax.broadcasted_iota(jnp.int32, (_K, _HW), 0)
    idx = jnp.min(jnp.where(d2 == m, kio, jnp.int32(_K)),
                  axis=0, keepdims=True)
    idx_ref[0] = idx
    loss_ref[0, 0, 0] = jnp.sum(m)


def _tc_stage(q, W):
    nb = q.shape[0]
    return pl.pallas_call(
        _tc_body,
        grid=(nb,),
        in_specs=[
            pl.BlockSpec((1, _C, _HW), lambda b: (b, 0, 0)),
            pl.BlockSpec((_K, _C), lambda b: (0, 0)),
        ],
        out_specs=[
            pl.BlockSpec((1, 1, _HW), lambda b: (b, 0, 0)),
            pl.BlockSpec((1, 1, 1), lambda b: (b, 0, 0), memory_space=pltpu.SMEM),
        ],
        out_shape=[
            jax.ShapeDtypeStruct((nb, 1, _HW), jnp.int32),
            jax.ShapeDtypeStruct((nb, 1, 1), jnp.float32),
        ],
    )(q, W)


_CPW = _C // 32


def _sc_gather(wt, idx):
    mesh = plsc.VectorSubcoreMesh(core_axis_name="c", subcore_axis_name="s")
    nb = idx.shape[0]

    @functools.partial(
        pl.kernel, mesh=mesh,
        compiler_params=pltpu.CompilerParams(needs_layout_passes=False),
        out_type=jax.ShapeDtypeStruct((nb, _C, _HW), jnp.float32),
        scratch_types=[
            pltpu.VMEM((_C, _K), jnp.float32),
            pltpu.VMEM((nb, _HW), jnp.int32),
            pltpu.VMEM((_CPW, _HW), jnp.float32),
            pltpu.VMEM((_CPW, _HW), jnp.float32),
            pltpu.SemaphoreType.DMA,
            pltpu.SemaphoreType.DMA,
        ],
    )
    def body(w_hbm, idx_hbm, out_hbm, w_v, idx_v, buf_a, buf_b, sem_a, sem_b):
        wid = lax.axis_index("s") * 2 + lax.axis_index("c")
        c0 = wid * _CPW
        pltpu.sync_copy(w_hbm, w_v)
        pltpu.sync_copy(idx_hbm, idx_v)

        bufs = (buf_a, buf_b)
        sems = (sem_a, sem_b)
        copies = [None, None]
        for b in range(nb):
            slot = b & 1
            if copies[slot] is not None:
                copies[slot].wait()
            buf = bufs[slot]

            def gather(j, _):
                v = idx_v[b, pl.ds(j * 16, 16)]
                vals = [plsc.load_gather(w_v.at[c0 + ci], [v])
                        for ci in range(_CPW)]
                for ci in range(_CPW):
                    buf[ci, pl.ds(j * 16, 16)] = vals[ci]
                return 0

            lax.fori_loop(0, _HW // 16, gather, 0, unroll=4)

            cp = pltpu.make_async_copy(
                buf, out_hbm.at[b, pl.ds(c0, _CPW)], sems[slot])
            cp.start()
            copies[slot] = cp
        for cp in copies:
            if cp is not None:
                cp.wait()

    return body(wt, idx)


def kernel(pre_quantized, W):
    q = pre_quantized.reshape(_B, _C, _HW)
    wt = W.T
    nc = 4
    cb = _B // nc
    outs, loss_parts = [], []
    for i in range(nc):
        idx, lp = _tc_stage(q[i * cb:(i + 1) * cb], W)
        idxs = idx.reshape(cb, 32, 32).swapaxes(1, 2).reshape(cb, _HW)
        outs.append(_sc_gather(wt, idxs))
        loss_parts.append(lp)
    out = jnp.concatenate(outs, axis=0)
    loss = jnp.sum(jnp.stack(loss_parts)) * (1.25 / (_B * _HW * _C))
    return out.reshape(_B, _C, 32, 32), loss


# speedup vs baseline: 1.2874x; 1.2874x over previous
import functools

import jax
import jax.numpy as jnp
from jax import lax
from jax.experimental import pallas as pl
from jax.experimental.pallas import tpu as pltpu
from jax.experimental.pallas import tpu_sc as plsc

_B, _C, _K, _HW = 16, 256, 128, 1024


def _tc_body(q_ref, w_ref, idx_ref, loss_ref):
    q = q_ref[0]
    w = w_ref[...]

    ab = lax.dot_general(w, q, (((1,), (0,)), ((), ())),
                         preferred_element_type=jnp.float32)
    b2 = jnp.sum(w * w, axis=1, keepdims=True)
    a2 = jnp.sum(q * q, axis=0, keepdims=True)
    d2 = jnp.maximum(a2 + b2 - 2.0 * ab, 0.0)

    m = jnp.min(d2, axis=0, keepdims=True)
    kio = lax.broadcasted_iota(jnp.int32, (_K, _HW), 0)
    idx = jnp.min(jnp.where(d2 == m, kio, jnp.int32(_K)),
                  axis=0, keepdims=True)
    idx_ref[0] = idx
    loss_ref[0, 0, 0] = jnp.sum(m)


def _tc_stage(q, W):
    nb = q.shape[0]
    return pl.pallas_call(
        _tc_body,
        grid=(nb,),
        in_specs=[
            pl.BlockSpec((1, _C, _HW), lambda b: (b, 0, 0)),
            pl.BlockSpec((_K, _C), lambda b: (0, 0)),
        ],
        out_specs=[
            pl.BlockSpec((1, 1, _HW), lambda b: (b, 0, 0)),
            pl.BlockSpec((1, 1, 1), lambda b: (b, 0, 0), memory_space=pltpu.SMEM),
        ],
        out_shape=[
            jax.ShapeDtypeStruct((nb, 1, _HW), jnp.int32),
            jax.ShapeDtypeStruct((nb, 1, 1), jnp.float32),
        ],
    )(q, W)


_CPW = _C // 32


def _sc_gather(wt, idx):
    mesh = plsc.VectorSubcoreMesh(core_axis_name="c", subcore_axis_name="s")
    nb = idx.shape[0]

    @functools.partial(
        pl.kernel, mesh=mesh,
        compiler_params=pltpu.CompilerParams(needs_layout_passes=False),
        out_type=jax.ShapeDtypeStruct((nb, _C, _HW), jnp.float32),
        scratch_types=[
            pltpu.VMEM((_C, _K), jnp.float32),
            pltpu.VMEM((nb, _HW), jnp.int32),
            pltpu.VMEM((_CPW, _HW), jnp.float32),
            pltpu.VMEM((_CPW, _HW), jnp.float32),
            pltpu.SemaphoreType.DMA,
            pltpu.SemaphoreType.DMA,
        ],
    )
    def body(w_hbm, idx_hbm, out_hbm, w_v, idx_v, buf_a, buf_b, sem_a, sem_b):
        wid = lax.axis_index("s") * 2 + lax.axis_index("c")
        c0 = wid * _CPW
        pltpu.sync_copy(w_hbm, w_v)
        pltpu.sync_copy(idx_hbm, idx_v)

        bufs = (buf_a, buf_b)
        sems = (sem_a, sem_b)
        copies = [None, None]
        for b in range(nb):
            slot = b & 1
            if copies[slot] is not None:
                copies[slot].wait()
            buf = bufs[slot]

            def gather(j, _):
                v = idx_v[b, pl.ds(j * 16, 16)]
                vals = [plsc.load_gather(w_v.at[c0 + ci], [v])
                        for ci in range(_CPW)]
                for ci in range(_CPW):
                    buf[ci, pl.ds(j * 16, 16)] = vals[ci]
                return 0

            lax.fori_loop(0, _HW // 16, gather, 0, unroll=8)

            cp = pltpu.make_async_copy(
                buf, out_hbm.at[b, pl.ds(c0, _CPW)], sems[slot])
            cp.start()
            copies[slot] = cp
        for cp in copies:
            if cp is not None:
                cp.wait()

    return body(wt, idx)


def kernel(pre_quantized, W):
    q = pre_quantized.reshape(_B, _C, _HW)
    idx, loss_parts = _tc_stage(q, W)
    idxs = idx.reshape(_B, 32, 32).swapaxes(1, 2).reshape(_B, _HW)
    out = _sc_gather(W.T, idxs)
    loss = jnp.sum(loss_parts) * (1.25 / (_B * _HW * _C))
    return out.reshape(_B, _C, 32, 32), loss


# speedup vs baseline: 1.3317x; 1.0344x over previous
import functools

import jax
import jax.numpy as jnp
from jax import lax
from jax.experimental import pallas as pl
from jax.experimental.pallas import tpu as pltpu
from jax.experimental.pallas import tpu_sc as plsc

_B, _C, _K, _HW = 16, 256, 128, 1024


def _tc_body(q_ref, w_ref, idx_ref, loss_ref):
    q = q_ref[0]
    w = w_ref[...]

    ab = lax.dot_general(w, q, (((1,), (0,)), ((), ())),
                         preferred_element_type=jnp.float32)
    b2 = jnp.sum(w * w, axis=1, keepdims=True)
    a2 = jnp.sum(q * q, axis=0, keepdims=True)
    d2 = jnp.maximum(a2 + b2 - 2.0 * ab, 0.0)

    m = jnp.min(d2, axis=0, keepdims=True)
    kio = lax.broadcasted_iota(jnp.int32, (_K, _HW), 0)
    idx = jnp.min(jnp.where(d2 == m, kio, jnp.int32(_K)),
                  axis=0, keepdims=True)
    idx_ref[0] = idx
    loss_ref[0, 0, 0] = jnp.sum(m)


def _tc_stage(q, W):
    nb = q.shape[0]
    return pl.pallas_call(
        _tc_body,
        grid=(nb,),
        in_specs=[
            pl.BlockSpec((1, _C, _HW), lambda b: (b, 0, 0)),
            pl.BlockSpec((_K, _C), lambda b: (0, 0)),
        ],
        out_specs=[
            pl.BlockSpec((1, 1, _HW), lambda b: (b, 0, 0)),
            pl.BlockSpec((1, 1, 1), lambda b: (b, 0, 0), memory_space=pltpu.SMEM),
        ],
        out_shape=[
            jax.ShapeDtypeStruct((nb, 1, _HW), jnp.int32),
            jax.ShapeDtypeStruct((nb, 1, 1), jnp.float32),
        ],
    )(q, W)


_CPW = _C // 32


def _sc_gather(wt, idx):
    mesh = plsc.VectorSubcoreMesh(core_axis_name="c", subcore_axis_name="s")
    nb = idx.shape[0]

    @functools.partial(
        pl.kernel, mesh=mesh,
        compiler_params=pltpu.CompilerParams(needs_layout_passes=False),
        out_type=jax.ShapeDtypeStruct((nb, _C, _HW), jnp.float32),
        scratch_types=[
            pltpu.VMEM((_CPW, _K), jnp.float32),
            pltpu.VMEM((nb, _HW), jnp.int32),
            pltpu.VMEM((_CPW, _HW), jnp.float32),
            pltpu.VMEM((_CPW, _HW), jnp.float32),
            pltpu.SemaphoreType.DMA,
            pltpu.SemaphoreType.DMA,
        ],
    )
    def body(w_hbm, idx_hbm, out_hbm, w_v, idx_v, buf_a, buf_b, sem_a, sem_b):
        wid = lax.axis_index("s") * 2 + lax.axis_index("c")
        c0 = wid * _CPW
        pltpu.sync_copy(w_hbm.at[pl.ds(c0, _CPW)], w_v)
        pltpu.sync_copy(idx_hbm, idx_v)

        bufs = (buf_a, buf_b)
        sems = (sem_a, sem_b)
        copies = [None, None]
        for b in range(nb):
            slot = b & 1
            if copies[slot] is not None:
                copies[slot].wait()
            buf = bufs[slot]

            def gather(j, _):
                v = idx_v[b, pl.ds(j * 16, 16)]
                vals = [plsc.load_gather(w_v.at[ci], [v])
                        for ci in range(_CPW)]
                for ci in range(_CPW):
                    buf[ci, pl.ds(j * 16, 16)] = vals[ci]
                return 0

            lax.fori_loop(0, _HW // 16, gather, 0, unroll=4)

            cp = pltpu.make_async_copy(
                buf, out_hbm.at[b, pl.ds(c0, _CPW)], sems[slot])
            cp.start()
            copies[slot] = cp
        for cp in copies:
            if cp is not None:
                cp.wait()

    return body(wt, idx)


def kernel(pre_quantized, W):
    q = pre_quantized.reshape(_B, _C, _HW)
    idx, loss_parts = _tc_stage(q, W)
    idxs = idx.reshape(_B, 32, 32).swapaxes(1, 2).reshape(_B, _HW)
    out = _sc_gather(W.T, idxs)
    loss = jnp.sum(loss_parts) * (1.25 / (_B * _HW * _C))
    return out.reshape(_B, _C, 32, 32), loss


# speedup vs baseline: 1.3948x; 1.0474x over previous
import functools

import jax
import jax.numpy as jnp
from jax import lax
from jax.experimental import pallas as pl
from jax.experimental.pallas import tpu as pltpu
from jax.experimental.pallas import tpu_sc as plsc

_B, _C, _K, _HW = 16, 256, 128, 1024


_BPS = 2


def _tc_body(q_ref, w_ref, idx_ref, loss_ref):
    w = w_ref[...]
    b2 = jnp.sum(w * w, axis=1, keepdims=True)
    for i in range(_BPS):
        q = q_ref[i]

        ab = lax.dot_general(w, q, (((1,), (0,)), ((), ())),
                             preferred_element_type=jnp.float32)
        a2 = jnp.sum(q * q, axis=0, keepdims=True)
        d2 = jnp.maximum(a2 + b2 - 2.0 * ab, 0.0)

        m = jnp.min(d2, axis=0, keepdims=True)
        kio = lax.broadcasted_iota(jnp.int32, (_K, _HW), 0)
        idx = jnp.min(jnp.where(d2 == m, kio, jnp.int32(_K)),
                      axis=0, keepdims=True)
        idx_ref[i] = idx
        loss_ref[i, 0, 0] = jnp.sum(m)


def _tc_stage(q, W):
    nb = q.shape[0]
    return pl.pallas_call(
        _tc_body,
        grid=(nb // _BPS,),
        in_specs=[
            pl.BlockSpec((_BPS, _C, _HW), lambda b: (b, 0, 0)),
            pl.BlockSpec((_K, _C), lambda b: (0, 0)),
        ],
        out_specs=[
            pl.BlockSpec((_BPS, 1, _HW), lambda b: (b, 0, 0)),
            pl.BlockSpec((_BPS, 1, 1), lambda b: (b, 0, 0),
                         memory_space=pltpu.SMEM),
        ],
        out_shape=[
            jax.ShapeDtypeStruct((nb, 1, _HW), jnp.int32),
            jax.ShapeDtypeStruct((nb, 1, 1), jnp.float32),
        ],
    )(q, W)


_CPW = _C // 32


def _sc_gather(wt, idx):
    mesh = plsc.VectorSubcoreMesh(core_axis_name="c", subcore_axis_name="s")
    nb = idx.shape[0]

    @functools.partial(
        pl.kernel, mesh=mesh,
        compiler_params=pltpu.CompilerParams(needs_layout_passes=False),
        out_type=jax.ShapeDtypeStruct((nb, _C, _HW), jnp.float32),
        scratch_types=[
            pltpu.VMEM((_CPW, _K), jnp.float32),
            pltpu.VMEM((nb, _HW), jnp.int32),
            pltpu.VMEM((_CPW, _HW), jnp.float32),
            pltpu.VMEM((_CPW, _HW), jnp.float32),
            pltpu.SemaphoreType.DMA,
            pltpu.SemaphoreType.DMA,
        ],
    )
    def body(w_hbm, idx_hbm, out_hbm, w_v, idx_v, buf_a, buf_b, sem_a, sem_b):
        wid = lax.axis_index("s") * 2 + lax.axis_index("c")
        c0 = wid * _CPW
        pltpu.sync_copy(w_hbm.at[pl.ds(c0, _CPW)], w_v)
        pltpu.sync_copy(idx_hbm, idx_v)

        bufs = (buf_a, buf_b)
        sems = (sem_a, sem_b)
        copies = [None, None]
        for b in range(nb):
            slot = b & 1
            if copies[slot] is not None:
                copies[slot].wait()
            buf = bufs[slot]

            def gather(j, _):
                v = idx_v[b, pl.ds(j * 16, 16)]
                vals = [plsc.load_gather(w_v.at[ci], [v])
                        for ci in range(_CPW)]
                for ci in range(_CPW):
                    buf[ci, pl.ds(j * 16, 16)] = vals[ci]
                return 0

            lax.fori_loop(0, _HW // 16, gather, 0, unroll=4)

            cp = pltpu.make_async_copy(
                buf, out_hbm.at[b, pl.ds(c0, _CPW)], sems[slot])
            cp.start()
            copies[slot] = cp
        for cp in copies:
            if cp is not None:
                cp.wait()

    return body(wt, idx)


def kernel(pre_quantized, W):
    q = pre_quantized.reshape(_B, _C, _HW)
    idx, loss_parts = _tc_stage(q, W)
    idxs = idx.reshape(_B, 32, 32).swapaxes(1, 2).reshape(_B, _HW)
    out = _sc_gather(W.T, idxs)
    loss = jnp.sum(loss_parts) * (1.25 / (_B * _HW * _C))
    return out.reshape(_B, _C, 32, 32), loss


# speedup vs baseline: 1.4170x; 1.0159x over previous
import functools

import jax
import jax.numpy as jnp
from jax import lax
from jax.experimental import pallas as pl
from jax.experimental.pallas import tpu as pltpu
from jax.experimental.pallas import tpu_sc as plsc

_B, _C, _K, _HW = 16, 256, 128, 1024


_BPS = 4


def _tc_body(q_ref, w_ref, idx_ref, loss_ref):
    w = w_ref[...]
    b2 = jnp.sum(w * w, axis=1, keepdims=True)
    for i in range(_BPS):
        q = q_ref[i]

        ab = lax.dot_general(w, q, (((1,), (0,)), ((), ())),
                             preferred_element_type=jnp.float32)
        a2 = jnp.sum(q * q, axis=0, keepdims=True)
        d2 = jnp.maximum(a2 + b2 - 2.0 * ab, 0.0)

        m = jnp.min(d2, axis=0, keepdims=True)
        kio = lax.broadcasted_iota(jnp.int32, (_K, _HW), 0)
        idx = jnp.min(jnp.where(d2 == m, kio, jnp.int32(_K)),
                      axis=0, keepdims=True)
        idx_ref[i] = idx
        loss_ref[i, 0, 0] = jnp.sum(m)


def _tc_stage(q, W):
    nb = q.shape[0]
    return pl.pallas_call(
        _tc_body,
        grid=(nb // _BPS,),
        in_specs=[
            pl.BlockSpec((_BPS, _C, _HW), lambda b: (b, 0, 0)),
            pl.BlockSpec((_K, _C), lambda b: (0, 0)),
        ],
        out_specs=[
            pl.BlockSpec((_BPS, 1, _HW), lambda b: (b, 0, 0)),
            pl.BlockSpec((_BPS, 1, 1), lambda b: (b, 0, 0),
                         memory_space=pltpu.SMEM),
        ],
        out_shape=[
            jax.ShapeDtypeStruct((nb, 1, _HW), jnp.int32),
            jax.ShapeDtypeStruct((nb, 1, 1), jnp.float32),
        ],
    )(q, W)


_CPW = _C // 32


def _sc_gather(wt, idx):
    mesh = plsc.VectorSubcoreMesh(core_axis_name="c", subcore_axis_name="s")
    nb = idx.shape[0]

    @functools.partial(
        pl.kernel, mesh=mesh,
        compiler_params=pltpu.CompilerParams(needs_layout_passes=False),
        out_type=jax.ShapeDtypeStruct((nb, _C, _HW), jnp.float32),
        scratch_types=[
            pltpu.VMEM((_CPW, _K), jnp.float32),
            pltpu.VMEM((nb, _HW), jnp.int32),
            pltpu.VMEM((_CPW, _HW), jnp.float32),
            pltpu.VMEM((_CPW, _HW), jnp.float32),
            pltpu.SemaphoreType.DMA,
            pltpu.SemaphoreType.DMA,
        ],
    )
    def body(w_hbm, idx_hbm, out_hbm, w_v, idx_v, buf_a, buf_b, sem_a, sem_b):
        wid = lax.axis_index("s") * 2 + lax.axis_index("c")
        c0 = wid * _CPW
        pltpu.sync_copy(w_hbm.at[pl.ds(c0, _CPW)], w_v)
        pltpu.sync_copy(idx_hbm, idx_v)

        bufs = (buf_a, buf_b)
        sems = (sem_a, sem_b)
        copies = [None, None]
        for b in range(nb):
            slot = b & 1
            if copies[slot] is not None:
                copies[slot].wait()
            buf = bufs[slot]

            def gather(j, _):
                v = idx_v[b, pl.ds(j * 16, 16)]
                vals = [plsc.load_gather(w_v.at[ci], [v])
                        for ci in range(_CPW)]
                for ci in range(_CPW):
                    buf[ci, pl.ds(j * 16, 16)] = vals[ci]
                return 0

            lax.fori_loop(0, _HW // 16, gather, 0, unroll=4)

            cp = pltpu.make_async_copy(
                buf, out_hbm.at[b, pl.ds(c0, _CPW)], sems[slot])
            cp.start()
            copies[slot] = cp
        for cp in copies:
            if cp is not None:
                cp.wait()

    return body(wt, idx)


def kernel(pre_quantized, W):
    q = pre_quantized.reshape(_B, _C, _HW)
    idx, loss_parts = _tc_stage(q, W)
    idxs = idx.reshape(_B, 32, 32).swapaxes(1, 2).reshape(_B, _HW)
    out = _sc_gather(W.T, idxs)
    loss = jnp.sum(loss_parts) * (1.25 / (_B * _HW * _C))
    return out.reshape(_B, _C, 32, 32), loss
